# baseline (speedup 1.0000x reference)
import functools

import jax
import jax.numpy as jnp
from jax import lax
from jax.experimental import pallas as pl
from jax.experimental.pallas import tpu as pltpu
from jax.experimental.pallas import tpu_sc as plsc

NUM_USERS = 200000
LANES = 16
CHUNK = 128


@functools.cache
def _build(B, D, V, NC, NS):
    NW = NC * NS
    b_per_w = B // NW
    n_chunks = b_per_w // CHUNK
    n_groups = b_per_w // LANES
    n_dchunks = D // LANES
    flat_per_w = b_per_w * D
    n_fchunks = flat_per_w // CHUNK
    rows_per_chunk = CHUNK // D
    FIRE = 8
    zero_row = 68
    mesh = plsc.VectorSubcoreMesh(core_axis_name="c", subcore_axis_name="s")

    @functools.partial(
        pl.kernel,
        mesh=mesh,
        compiler_params=pltpu.CompilerParams(
            needs_layout_passes=False, use_tc_tiling_on_sc=False),
        out_type=jax.ShapeDtypeStruct((B * D,), jnp.float32),
        scratch_types=[
            pltpu.VMEM((b_per_w,), jnp.int32),
            pltpu.VMEM((4, b_per_w), jnp.int32),
            pltpu.VMEM((flat_per_w,), jnp.int32),
            pltpu.VMEM((flat_per_w,), jnp.float32),
            pltpu.VMEM((69, D), jnp.float32),
            pltpu.SemaphoreType.DMA,
        ],
    )
    def sc_kernel(e_hbm, x3_hbm, x4_hbm, x5_hbm, x6_hbm, ft_hbm, ntf_hbm,
                  out_hbm, e_v, xc_v, ni_v, acc_v, ft_v, sem):
        wid = lax.axis_index("s") * NC + lax.axis_index("c")
        base = wid * b_per_w

        pltpu.sync_copy(e_hbm.at[wid], e_v)
        pltpu.sync_copy(ft_hbm, ft_v)

        descs = []
        for k in range(n_chunks):
            idx = e_v.at[pl.ds(k * CHUNK, CHUNK)]
            for c, xh in enumerate((x3_hbm, x4_hbm, x5_hbm, x6_hbm)):
                descs.append(pltpu.async_copy(
                    xh.at[idx], xc_v.at[c, pl.ds(k * CHUNK, CHUNK)], sem))

        iota = lax.iota(jnp.int32, LANES)
        ramps = [(c * LANES + iota) * V for c in range(n_dchunks)]

        def build_idx(g, carry):
            gbase = g * LANES
            ev = e_v[pl.ds(gbase, LANES)]
            for l in range(LANES):
                er = ev[l]
                fb = (gbase + l) * D
                for c in range(n_dchunks):
                    ni_v[pl.ds(fb + c * LANES, LANES)] = ramps[c] + er
            return carry

        lax.fori_loop(0, n_groups, build_idx, 0)

        def fire_batch(s, carry):
            bds = []
            for b in range(FIRE):
                sl = pl.ds((s * FIRE + b) * CHUNK, CHUNK)
                bds.append(pltpu.async_copy(
                    ntf_hbm.at[ni_v.at[sl]], acc_v.at[sl], sem))
            for d in bds:
                d.wait()
            return carry

        lax.fori_loop(0, n_fchunks // FIRE, fire_batch, 0)
        for dsc in descs:
            dsc.wait()

        def group(g, carry):
            gbase = g * LANES
            ev = e_v[pl.ds(gbase, LANES)]
            mask = ev >= NUM_USERS
            f3 = jnp.where(mask, xc_v[0, pl.ds(gbase, LANES)] + 46, zero_row)
            f4 = jnp.where(mask, xc_v[1, pl.ds(gbase, LANES)] + 35, zero_row)
            f5 = jnp.where(mask, xc_v[2, pl.ds(gbase, LANES)], zero_row)
            f6 = jnp.where(mask, xc_v[3, pl.ds(gbase, LANES)] + 9, zero_row)
            for l in range(LANES):
                fb = (gbase + l) * D
                s3, s4, s5, s6 = f3[l], f4[l], f5[l], f6[l]
                for c in range(n_dchunks):
                    dcol = pl.ds(c * LANES, LANES)
                    fsl = pl.ds(fb + c * LANES, LANES)
                    acc_v[fsl] = (acc_v[fsl]
                                  + ft_v[s5, dcol] + ft_v[s6, dcol]
                                  + ft_v[s4, dcol] + ft_v[s3, dcol])
            return carry

        lax.fori_loop(0, n_groups, group, 0)
        pltpu.sync_copy(acc_v, out_hbm.at[pl.ds(base * D, flat_per_w)])

    return sc_kernel


def kernel(e, x, feat_table, name_emb):
    B = e.shape[0]
    V, D = name_emb.shape
    info = plsc.get_sparse_core_info()
    NC, NS = info.num_cores, info.num_subcores
    NW = NC * NS
    e2 = e.astype(jnp.int32).reshape(NW, B // NW)
    xi = x.astype(jnp.int32)
    x3, x4, x5, x6 = xi[:, 3], xi[:, 4], xi[:, 5], xi[:, 6]
    ftp = jnp.concatenate(
        [feat_table, jnp.zeros((1, D), feat_table.dtype)], axis=0)
    ntf = jnp.transpose(name_emb).reshape(V * D)
    flat = _build(B, D, V, NC, NS)(e2, x3, x4, x5, x6, ftp, ntf)
    return flat.reshape(B, D)


# speedup vs baseline: 10.1940x; 10.1940x over previous
import functools

import jax
import jax.numpy as jnp
from jax import lax
from jax.experimental import pallas as pl
from jax.experimental.pallas import tpu as pltpu
from jax.experimental.pallas import tpu_sc as plsc

NUM_USERS = 200000
LANES = 16
CHUNK = 128


@functools.cache
def _build(B, D, NC, NS):
    NW = NC * NS
    b_per_w = B // NW
    n_chunks = b_per_w // CHUNK
    n_groups = b_per_w // LANES
    n_dchunks = D // LANES
    flat_per_w = b_per_w * D
    QR = 64
    zero_off = 68 * D
    mesh = plsc.VectorSubcoreMesh(core_axis_name="c", subcore_axis_name="s")

    @functools.partial(
        pl.kernel,
        mesh=mesh,
        compiler_params=pltpu.CompilerParams(
            needs_layout_passes=False, use_tc_tiling_on_sc=True),
        out_type=jax.ShapeDtypeStruct((B * D,), jnp.float32),
        scratch_types=[
            pltpu.VMEM((b_per_w,), jnp.int32),
            pltpu.VMEM((b_per_w,), jnp.int32),
            pltpu.VMEM((b_per_w,), jnp.int32),
            pltpu.VMEM((b_per_w,), jnp.int32),
            pltpu.VMEM((b_per_w,), jnp.int32),
            pltpu.VMEM((QR, 8, D), jnp.float32),
            pltpu.VMEM((flat_per_w,), jnp.float32),
            pltpu.VMEM((69 * D,), jnp.float32),
            pltpu.SemaphoreType.DMA,
            pltpu.SemaphoreType.DMA,
        ],
    )
    def sc_kernel(e_hbm, x3_hbm, x4_hbm, x5_hbm, x6_hbm, ft_hbm, name_hbm,
                  out_hbm, e_v, c3_v, c4_v, c5_v, c6_v, st_v, acc_v, ft_v,
                  sem, nsem):
        wid = lax.axis_index("s") * NC + lax.axis_index("c")
        base = wid * b_per_w

        pltpu.sync_copy(e_hbm.at[pl.ds(base, b_per_w)], e_v)
        pltpu.sync_copy(ft_hbm, ft_v)

        descs = []
        for k in range(n_chunks):
            idx = e_v.at[pl.ds(k * CHUNK, CHUNK)]
            sl = pl.ds(k * CHUNK, CHUNK)
            for xh, cv in ((x3_hbm, c3_v), (x4_hbm, c4_v),
                           (x5_hbm, c5_v), (x6_hbm, c6_v)):
                descs.append(pltpu.async_copy(xh.at[idx], cv.at[sl], sem))
        for dsc in descs:
            dsc.wait()

        for q in range(b_per_w // QR):
            qbase = q * QR

            def qfire(g, carry):
                ev = e_v[pl.ds(qbase + g * LANES, LANES)]
                tb = lax.shift_left(lax.shift_right_logical(ev, 3), 3)
                nds = []
                for l in range(LANES):
                    tbl = pl.multiple_of(tb[l], 8)
                    nds.append(pltpu.async_copy(
                        name_hbm.at[pl.ds(tbl, 8), :],
                        st_v.at[g * LANES + l], nsem))
                for dsc in nds:
                    dsc.wait()
                return carry

            lax.fori_loop(0, QR // LANES, qfire, 0)

            def qgroup(g, carry):
                gbase = qbase + g * LANES
                ev = e_v[pl.ds(gbase, LANES)]
                mask = ev >= NUM_USERS
                sub = ev & 7
                f3 = jnp.where(mask, (c3_v[pl.ds(gbase, LANES)] + 46) * D,
                               zero_off)
                f4 = jnp.where(mask, (c4_v[pl.ds(gbase, LANES)] + 35) * D,
                               zero_off)
                f5 = jnp.where(mask, c5_v[pl.ds(gbase, LANES)] * D, zero_off)
                f6 = jnp.where(mask, (c6_v[pl.ds(gbase, LANES)] + 9) * D,
                               zero_off)
                for l in range(LANES):
                    fb = (gbase + l) * D
                    j = g * LANES + l
                    s3, s4, s5, s6 = f3[l], f4[l], f5[l], f6[l]
                    sb = sub[l]
                    for c in range(n_dchunks):
                        cl = c * LANES
                        acc_v[pl.ds(fb + cl, LANES)] = (
                            st_v[j, sb, pl.ds(cl, LANES)]
                            + ft_v[pl.ds(s5 + cl, LANES)]
                            + ft_v[pl.ds(s6 + cl, LANES)]
                            + ft_v[pl.ds(s4 + cl, LANES)]
                            + ft_v[pl.ds(s3 + cl, LANES)])
                return carry

            lax.fori_loop(0, QR // LANES, qgroup, 0)

        pltpu.sync_copy(acc_v, out_hbm.at[pl.ds(base * D, flat_per_w)])

    return sc_kernel


def kernel(e, x, feat_table, name_emb):
    B = e.shape[0]
    D = feat_table.shape[1]
    info = plsc.get_sparse_core_info()
    NC, NS = info.num_cores, info.num_subcores
    e1 = e.astype(jnp.int32)
    xi = x.astype(jnp.int32)
    x3, x4, x5, x6 = xi[:, 3], xi[:, 4], xi[:, 5], xi[:, 6]
    ftp = jnp.concatenate(
        [feat_table, jnp.zeros((1, D), feat_table.dtype)], axis=0)
    ftf = ftp.reshape(69 * D)
    flat = _build(B, D, NC, NS)(e1, x3, x4, x5, x6, ftf, name_emb)
    return flat.reshape(B, D)


# speedup vs baseline: 10.5436x; 1.0343x over previous
import functools

import jax
import jax.numpy as jnp
from jax import lax
from jax.experimental import pallas as pl
from jax.experimental.pallas import tpu as pltpu
from jax.experimental.pallas import tpu_sc as plsc

NUM_USERS = 200000
LANES = 16
CHUNK = 128


@functools.cache
def _build(B, D, NC, NS):
    NW = NC * NS
    b_per_w = B // NW
    n_chunks = b_per_w // CHUNK
    n_groups = b_per_w // LANES
    n_dchunks = D // LANES
    flat_per_w = b_per_w * D
    QR = 64
    zero_off = 68 * D
    mesh = plsc.VectorSubcoreMesh(core_axis_name="c", subcore_axis_name="s")

    @functools.partial(
        pl.kernel,
        mesh=mesh,
        compiler_params=pltpu.CompilerParams(
            needs_layout_passes=False, use_tc_tiling_on_sc=True),
        out_type=jax.ShapeDtypeStruct((B * D,), jnp.float32),
        scratch_types=[
            pltpu.VMEM((b_per_w,), jnp.int32),
            pltpu.VMEM((b_per_w,), jnp.int32),
            pltpu.VMEM((b_per_w,), jnp.int32),
            pltpu.VMEM((b_per_w,), jnp.int32),
            pltpu.VMEM((b_per_w,), jnp.int32),
            pltpu.VMEM((QR * 8, D), jnp.float32),
            pltpu.VMEM((flat_per_w,), jnp.float32),
            pltpu.VMEM((69 * D,), jnp.float32),
            pltpu.SemaphoreType.DMA,
            pltpu.SemaphoreType.DMA,
        ],
    )
    def sc_kernel(e_hbm, x3_hbm, x4_hbm, x5_hbm, x6_hbm, ft_hbm, name_hbm,
                  out_hbm, e_v, c3_v, c4_v, c5_v, c6_v, st_v, acc_v, ft_v,
                  sem, nsem):
        wid = lax.axis_index("s") * NC + lax.axis_index("c")
        base = wid * b_per_w

        pltpu.sync_copy(e_hbm.at[pl.ds(base, b_per_w)], e_v)
        pltpu.sync_copy(ft_hbm, ft_v)

        descs = []
        for k in range(n_chunks):
            idx = e_v.at[pl.ds(k * CHUNK, CHUNK)]
            sl = pl.ds(k * CHUNK, CHUNK)
            for xh, cv in ((x3_hbm, c3_v), (x4_hbm, c4_v),
                           (x5_hbm, c5_v), (x6_hbm, c6_v)):
                descs.append(pltpu.async_copy(xh.at[idx], cv.at[sl], sem))
        for dsc in descs:
            dsc.wait()

        for q in range(b_per_w // QR):
            qbase = q * QR

            def qfire(g, carry):
                ev = e_v[pl.ds(qbase + g * LANES, LANES)]
                tb = lax.shift_left(lax.shift_right_logical(ev, 3), 3)
                for l in range(LANES):
                    tbl = pl.multiple_of(tb[l], 8)
                    dst = pl.multiple_of((g * LANES + l) * 8, 8)
                    pltpu.async_copy(
                        name_hbm.at[pl.ds(tbl, 8), :],
                        st_v.at[pl.ds(dst, 8), :], nsem)
                return carry

            lax.fori_loop(0, QR // LANES, qfire, 0)
            pltpu.make_async_copy(
                name_hbm.at[pl.ds(0, QR * 8), :], st_v, nsem).wait()

            def qgroup(g, carry):
                gbase = qbase + g * LANES
                ev = e_v[pl.ds(gbase, LANES)]
                mask = ev >= NUM_USERS
                sub = ev & 7
                f3 = jnp.where(mask, (c3_v[pl.ds(gbase, LANES)] + 46) * D,
                               zero_off)
                f4 = jnp.where(mask, (c4_v[pl.ds(gbase, LANES)] + 35) * D,
                               zero_off)
                f5 = jnp.where(mask, c5_v[pl.ds(gbase, LANES)] * D, zero_off)
                f6 = jnp.where(mask, (c6_v[pl.ds(gbase, LANES)] + 9) * D,
                               zero_off)
                for l in range(LANES):
                    fb = (gbase + l) * D
                    jr = (g * LANES + l) * 8 + sub[l]
                    s3, s4, s5, s6 = f3[l], f4[l], f5[l], f6[l]
                    for c in range(n_dchunks):
                        cl = c * LANES
                        acc_v[pl.ds(fb + cl, LANES)] = (
                            st_v[jr, pl.ds(cl, LANES)]
                            + ft_v[pl.ds(s5 + cl, LANES)]
                            + ft_v[pl.ds(s6 + cl, LANES)]
                            + ft_v[pl.ds(s4 + cl, LANES)]
                            + ft_v[pl.ds(s3 + cl, LANES)])
                return carry

            lax.fori_loop(0, QR // LANES, qgroup, 0)

        pltpu.sync_copy(acc_v, out_hbm.at[pl.ds(base * D, flat_per_w)])

    return sc_kernel


def kernel(e, x, feat_table, name_emb):
    B = e.shape[0]
    D = feat_table.shape[1]
    info = plsc.get_sparse_core_info()
    NC, NS = info.num_cores, info.num_subcores
    e1 = e.astype(jnp.int32)
    xi = x.astype(jnp.int32)
    x3, x4, x5, x6 = xi[:, 3], xi[:, 4], xi[:, 5], xi[:, 6]
    ftp = jnp.concatenate(
        [feat_table, jnp.zeros((1, D), feat_table.dtype)], axis=0)
    ftf = ftp.reshape(69 * D)
    flat = _build(B, D, NC, NS)(e1, x3, x4, x5, x6, ftf, name_emb)
    return flat.reshape(B, D)


# speedup vs baseline: 10.8125x; 1.0255x over previous
import functools

import jax
import jax.numpy as jnp
from jax import lax
from jax.experimental import pallas as pl
from jax.experimental.pallas import tpu as pltpu
from jax.experimental.pallas import tpu_sc as plsc

NUM_USERS = 200000
LANES = 16
CHUNK = 128


@functools.cache
def _build_feat(B, D, NC, NS):
    NW = NC * NS
    b_per_w = B // NW
    n_chunks = b_per_w // CHUNK
    n_groups = b_per_w // LANES
    n_dchunks = D // LANES
    flat_per_w = b_per_w * D
    zero_off = 68 * D
    mesh = plsc.VectorSubcoreMesh(core_axis_name="c", subcore_axis_name="s")

    @functools.partial(
        pl.kernel,
        mesh=mesh,
        compiler_params=pltpu.CompilerParams(
            needs_layout_passes=False, use_tc_tiling_on_sc=False),
        out_type=jax.ShapeDtypeStruct((B * D,), jnp.float32),
        scratch_types=[
            pltpu.VMEM((b_per_w,), jnp.int32),
            pltpu.VMEM((b_per_w,), jnp.int32),
            pltpu.VMEM((b_per_w,), jnp.int32),
            pltpu.VMEM((b_per_w,), jnp.int32),
            pltpu.VMEM((b_per_w,), jnp.int32),
            pltpu.VMEM((flat_per_w,), jnp.float32),
            pltpu.VMEM((69 * D,), jnp.float32),
            pltpu.SemaphoreType.DMA,
        ],
    )
    def ka(e_hbm, x3_hbm, x4_hbm, x5_hbm, x6_hbm, ft_hbm,
           out_hbm, e_v, c3_v, c4_v, c5_v, c6_v, acc_v, ft_v, sem):
        wid = lax.axis_index("s") * NC + lax.axis_index("c")
        base = wid * b_per_w

        pltpu.sync_copy(e_hbm.at[pl.ds(base, b_per_w)], e_v)
        pltpu.sync_copy(ft_hbm, ft_v)

        descs = []
        for k in range(n_chunks):
            idx = e_v.at[pl.ds(k * CHUNK, CHUNK)]
            sl = pl.ds(k * CHUNK, CHUNK)
            for xh, cv in ((x3_hbm, c3_v), (x4_hbm, c4_v),
                           (x5_hbm, c5_v), (x6_hbm, c6_v)):
                descs.append(pltpu.async_copy(xh.at[idx], cv.at[sl], sem))
        for dsc in descs:
            dsc.wait()

        def group(g, carry):
            gbase = g * LANES
            ev = e_v[pl.ds(gbase, LANES)]
            mask = ev >= NUM_USERS
            f3 = jnp.where(mask, (c3_v[pl.ds(gbase, LANES)] + 46) * D,
                           zero_off)
            f4 = jnp.where(mask, (c4_v[pl.ds(gbase, LANES)] + 35) * D,
                           zero_off)
            f5 = jnp.where(mask, c5_v[pl.ds(gbase, LANES)] * D, zero_off)
            f6 = jnp.where(mask, (c6_v[pl.ds(gbase, LANES)] + 9) * D,
                           zero_off)
            for l in range(LANES):
                fb = (gbase + l) * D
                s3, s4, s5, s6 = f3[l], f4[l], f5[l], f6[l]
                for c in range(n_dchunks):
                    cl = c * LANES
                    acc_v[pl.ds(fb + cl, LANES)] = (
                        ft_v[pl.ds(s5 + cl, LANES)]
                        + ft_v[pl.ds(s6 + cl, LANES)]
                        + ft_v[pl.ds(s4 + cl, LANES)]
                        + ft_v[pl.ds(s3 + cl, LANES)])
            return carry

        lax.fori_loop(0, n_groups, group, 0)
        pltpu.sync_copy(acc_v, out_hbm.at[pl.ds(base * D, flat_per_w)])

    return ka


@functools.cache
def _build_name_add(B, D, NC, NS):
    NW = NC * NS
    b_per_w = B // NW
    n_groups = b_per_w // LANES
    n_dchunks = D // LANES
    flat_per_w = b_per_w * D
    QR = 64
    mesh = plsc.VectorSubcoreMesh(core_axis_name="c", subcore_axis_name="s")

    @functools.partial(
        pl.kernel,
        mesh=mesh,
        compiler_params=pltpu.CompilerParams(
            needs_layout_passes=False, use_tc_tiling_on_sc=True),
        out_type=jax.ShapeDtypeStruct((B * D,), jnp.float32),
        scratch_types=[
            pltpu.VMEM((b_per_w,), jnp.int32),
            pltpu.VMEM((QR * 8, D), jnp.float32),
            pltpu.VMEM((flat_per_w,), jnp.float32),
            pltpu.SemaphoreType.DMA,
        ],
    )
    def kb(e_hbm, parts_hbm, name_hbm, out_hbm, e_v, st_v, acc_v, nsem):
        wid = lax.axis_index("s") * NC + lax.axis_index("c")
        base = wid * b_per_w

        pltpu.sync_copy(e_hbm.at[pl.ds(base, b_per_w)], e_v)
        pltpu.sync_copy(parts_hbm.at[pl.ds(base * D, flat_per_w)], acc_v)

        for q in range(b_per_w // QR):
            qbase = q * QR

            def qfire(g, carry):
                ev = e_v[pl.ds(qbase + g * LANES, LANES)]
                tb = lax.shift_left(lax.shift_right_logical(ev, 3), 3)
                for l in range(LANES):
                    tbl = pl.multiple_of(tb[l], 8)
                    dst = pl.multiple_of((g * LANES + l) * 8, 8)
                    pltpu.async_copy(
                        name_hbm.at[pl.ds(tbl, 8), :],
                        st_v.at[pl.ds(dst, 8), :], nsem)
                return carry

            lax.fori_loop(0, QR // LANES, qfire, 0)
            pltpu.make_async_copy(
                name_hbm.at[pl.ds(0, QR * 8), :], st_v, nsem).wait()

            def qgroup(g, carry):
                gbase = qbase + g * LANES
                ev = e_v[pl.ds(gbase, LANES)]
                sub = ev & 7
                for l in range(LANES):
                    fb = (gbase + l) * D
                    jr = (g * LANES + l) * 8 + sub[l]
                    for c in range(n_dchunks):
                        cl = c * LANES
                        fsl = pl.ds(fb + cl, LANES)
                        acc_v[fsl] = acc_v[fsl] + st_v[jr, pl.ds(cl, LANES)]
                return carry

            lax.fori_loop(0, QR // LANES, qgroup, 0)

        pltpu.sync_copy(acc_v, out_hbm.at[pl.ds(base * D, flat_per_w)])

    return kb


def kernel(e, x, feat_table, name_emb):
    B = e.shape[0]
    D = feat_table.shape[1]
    info = plsc.get_sparse_core_info()
    NC, NS = info.num_cores, info.num_subcores
    e1 = e.astype(jnp.int32)
    xi = x.astype(jnp.int32)
    x3, x4, x5, x6 = xi[:, 3], xi[:, 4], xi[:, 5], xi[:, 6]
    ftp = jnp.concatenate(
        [feat_table, jnp.zeros((1, D), feat_table.dtype)], axis=0)
    ftf = ftp.reshape(69 * D)
    parts = _build_feat(B, D, NC, NS)(e1, x3, x4, x5, x6, ftf)
    flat = _build_name_add(B, D, NC, NS)(e1, parts, name_emb)
    return flat.reshape(B, D)
